# Initial kernel scaffold; baseline (speedup 1.0000x reference)
import jax
import jax.numpy as jnp
from jax.experimental import pallas as pl


def kernel(x, edge_index, batch, W1, b1, W2, b2, Wout, bout):
    raise NotImplementedError("write your pallas kernel here")


# speedup vs baseline: 11.4306x; 11.4306x over previous
import functools

import jax
import jax.numpy as jnp
from jax import lax
from jax.experimental import pallas as pl
from jax.experimental.pallas import tpu as pltpu
from jax.experimental.pallas import tpu_sc as plsc

N = 10000
D = 128
E = 320000
G = 64

NC = 2
NS = 16
NW = NC * NS
CH = 128
EPT = E // NW
K = (EPT + CH - 1) // CH
EPT_PAD = K * CH
E_PAD = NW * EPT_PAD
SCAT_ROWS = 10240
RPT = SCAT_ROWS // NS
ZB = 128
TRASH = N

DEGW = 128

RB = 1000
NBLK = N // RB

_sc_mesh = plsc.VectorSubcoreMesh(core_axis_name="c", subcore_axis_name="s")


@functools.partial(
    pl.kernel,
    out_type=jax.ShapeDtypeStruct((NC, SCAT_ROWS, DEGW), jnp.float32),
    mesh=_sc_mesh,
    scratch_types=[
        pltpu.VMEM((K, CH), jnp.int32),
        pltpu.VMEM((CH, DEGW), jnp.float32),
        pltpu.VMEM_SHARED((SCAT_ROWS, DEGW), jnp.float32),
    ],
)
def _deg_kernel(dst3, ones_col, zeros_col, out, idx_v, ones_v, deg_sh):
    c = lax.axis_index("c")
    s = lax.axis_index("s")
    wid = c * NS + s
    pltpu.sync_copy(zeros_col, deg_sh.at[pl.ds(s * RPT, RPT)])
    pltpu.sync_copy(ones_col, ones_v)
    pltpu.sync_copy(dst3.at[wid], idx_v)
    plsc.subcore_barrier()

    def body(j, carry):
        pltpu.sync_copy(ones_v, deg_sh.at[idx_v.at[j]], add=True)
        return carry

    lax.fori_loop(0, K, body, 0)
    plsc.subcore_barrier()
    pltpu.sync_copy(deg_sh.at[pl.ds(s * RPT, RPT)], out.at[c, pl.ds(s * RPT, RPT)])


@functools.partial(
    pl.kernel,
    out_type=jax.ShapeDtypeStruct((NC, SCAT_ROWS, D), jnp.float32),
    mesh=_sc_mesh,
    scratch_types=[
        pltpu.VMEM((K, CH), jnp.int32),
        pltpu.VMEM((K, CH), jnp.int32),
        pltpu.VMEM((CH, D), jnp.float32),
        pltpu.VMEM_SHARED((SCAT_ROWS, D), jnp.float32),
    ],
)
def _edge_kernel(u, src3, dst3, zeros_blk, out, src_v, dst_v, rows_v, scat_sh):
    c = lax.axis_index("c")
    s = lax.axis_index("s")
    wid = c * NS + s
    for t in range(RPT // ZB):
        pltpu.sync_copy(zeros_blk, scat_sh.at[pl.ds(s * RPT + t * ZB, ZB)])
    pltpu.sync_copy(src3.at[wid], src_v)
    pltpu.sync_copy(dst3.at[wid], dst_v)
    plsc.subcore_barrier()

    def body(j, carry):
        pltpu.sync_copy(u.at[src_v.at[j]], rows_v)
        pltpu.sync_copy(rows_v, scat_sh.at[dst_v.at[j]], add=True)
        return carry

    lax.fori_loop(0, K, body, 0)
    plsc.subcore_barrier()
    pltpu.sync_copy(scat_sh.at[pl.ds(s * RPT, RPT)], out.at[c, pl.ds(s * RPT, RPT)])


def _tc1_body(x_ref, w_ref, da_ref, db_ref, u_ref, dinv_ref):
    deg = da_ref[...] + db_ref[...] + 1.0
    dinv = lax.rsqrt(deg)
    xw = jnp.dot(x_ref[...], w_ref[...], preferred_element_type=jnp.float32)
    u_ref[...] = xw * dinv
    dinv_ref[...] = dinv


def _tc1(x, W1, dega, degb):
    return pl.pallas_call(
        _tc1_body,
        grid=(NBLK,),
        in_specs=[
            pl.BlockSpec((RB, D), lambda i: (i, 0)),
            pl.BlockSpec((D, D), lambda i: (0, 0)),
            pl.BlockSpec((RB, 1), lambda i: (i, 0)),
            pl.BlockSpec((RB, 1), lambda i: (i, 0)),
        ],
        out_specs=[
            pl.BlockSpec((RB, D), lambda i: (i, 0)),
            pl.BlockSpec((RB, 1), lambda i: (i, 0)),
        ],
        out_shape=[
            jax.ShapeDtypeStruct((N, D), jnp.float32),
            jax.ShapeDtypeStruct((N, 1), jnp.float32),
        ],
    )(x, W1, dega, degb)


def _tc2_body(sa_ref, sb_ref, u_ref, dinv_ref, b_ref, w_ref, h_ref, u2_ref):
    dv = dinv_ref[...]
    h = jnp.maximum((sa_ref[...] + sb_ref[...] + u_ref[...]) * dv + b_ref[...], 0.0)
    h_ref[...] = h
    u2_ref[...] = jnp.dot(h, w_ref[...], preferred_element_type=jnp.float32) * dv


def _tc2(sa, sb, u1, dinv, b1, W2):
    return pl.pallas_call(
        _tc2_body,
        grid=(NBLK,),
        in_specs=[
            pl.BlockSpec((RB, D), lambda i: (i, 0)),
            pl.BlockSpec((RB, D), lambda i: (i, 0)),
            pl.BlockSpec((RB, D), lambda i: (i, 0)),
            pl.BlockSpec((RB, 1), lambda i: (i, 0)),
            pl.BlockSpec((1, D), lambda i: (0, 0)),
            pl.BlockSpec((D, D), lambda i: (0, 0)),
        ],
        out_specs=[
            pl.BlockSpec((RB, D), lambda i: (i, 0)),
            pl.BlockSpec((RB, D), lambda i: (i, 0)),
        ],
        out_shape=[
            jax.ShapeDtypeStruct((N, D), jnp.float32),
            jax.ShapeDtypeStruct((N, D), jnp.float32),
        ],
    )(sa, sb, u1, dinv, b1, W2)


def _tc3_body(sa_ref, sb_ref, u2_ref, dinv_ref, b2_ref, h1_ref, batch_ref,
              wout_ref, bout_ref, out_ref):
    h2 = ((sa_ref[...] + sb_ref[...] + u2_ref[...]) * dinv_ref[...]
          + b2_ref[...] + h1_ref[...])
    bt = batch_ref[...]
    ids = lax.broadcasted_iota(jnp.int32, (G, N), 0)
    p = (ids == jnp.broadcast_to(bt, (G, N))).astype(jnp.float32)
    sums = jnp.dot(p, h2, preferred_element_type=jnp.float32)
    cnt = jnp.sum(p, axis=1, keepdims=True)
    gm = sums / jnp.maximum(cnt, 1.0)
    out_ref[...] = lax.dot_general(
        gm, wout_ref[...], (((1,), (1,)), ((), ())),
        preferred_element_type=jnp.float32) + bout_ref[...]


def _tc3(sa, sb, u2, dinv, b2, h1, batch2d, Wout, bout):
    return pl.pallas_call(
        _tc3_body,
        out_shape=jax.ShapeDtypeStruct((G, D), jnp.float32),
    )(sa, sb, u2, dinv, b2, h1, batch2d, Wout, bout)


def kernel(x, edge_index, batch, W1, b1, W2, b2, Wout, bout):
    src = edge_index[0].astype(jnp.int32)
    dst = edge_index[1].astype(jnp.int32)
    pad = E_PAD - E
    src3 = jnp.concatenate([src, jnp.zeros((pad,), jnp.int32)]).reshape(NW, K, CH)
    dst3 = jnp.concatenate([dst, jnp.full((pad,), TRASH, jnp.int32)]).reshape(NW, K, CH)
    ones_col = jnp.ones((CH, DEGW), jnp.float32)
    zeros_col = jnp.zeros((RPT, DEGW), jnp.float32)
    zeros_blk = jnp.zeros((ZB, D), jnp.float32)

    deg = _deg_kernel(dst3, ones_col, zeros_col)
    dega = deg[0, :N, 0:1]
    degb = deg[1, :N, 0:1]
    u1, dinv = _tc1(x, W1, dega, degb)
    s1 = _edge_kernel(u1, src3, dst3, zeros_blk)
    h1, u2 = _tc2(s1[0, :N], s1[1, :N], u1, dinv, b1.reshape(1, D), W2)
    s2 = _edge_kernel(u2, src3, dst3, zeros_blk)
    out = _tc3(s2[0, :N], s2[1, :N], u2, dinv, b2.reshape(1, D), h1,
               batch.reshape(1, N).astype(jnp.int32), Wout,
               bout.reshape(1, D))
    return out


# speedup vs baseline: 11.7913x; 1.0316x over previous
import functools

import jax
import jax.numpy as jnp
from jax import lax
from jax.experimental import pallas as pl
from jax.experimental.pallas import tpu as pltpu
from jax.experimental.pallas import tpu_sc as plsc

N = 10000
D = 128
E = 320000
G = 64

NC = 2
NS = 16
NW = NC * NS
CH = 96
EPT = E // NW
NBUF = 2
K = 106
EPT_PAD = K * CH
E_PAD = NW * EPT_PAD
SCAT_ROWS = 10112
RPT = SCAT_ROWS // NS
NZB = 1
ZB = RPT // NZB
TRASH = N

DEGW = 128

RB = 1000
NBLK = N // RB

_sc_mesh = plsc.VectorSubcoreMesh(core_axis_name="c", subcore_axis_name="s")


@functools.partial(
    pl.kernel,
    out_type=jax.ShapeDtypeStruct((NC, SCAT_ROWS, DEGW), jnp.float32),
    mesh=_sc_mesh,
    scratch_types=[
        pltpu.VMEM((K, CH), jnp.int32),
        pltpu.VMEM((CH, DEGW), jnp.float32),
        pltpu.VMEM_SHARED((SCAT_ROWS, DEGW), jnp.float32),
        pltpu.SemaphoreType.DMA,
    ],
)
def _deg_kernel(dst3, ones_col, zeros_col, out, idx_v, ones_v, deg_sh, dsem):
    c = lax.axis_index("c")
    s = lax.axis_index("s")
    wid = c * NS + s
    pltpu.sync_copy(zeros_col, deg_sh.at[pl.ds(s * RPT, RPT)])
    pltpu.sync_copy(ones_col, ones_v)
    pltpu.sync_copy(dst3.at[wid], idx_v)
    plsc.subcore_barrier()

    def fire(j, carry):
        pltpu.async_copy(ones_v, deg_sh.at[idx_v.at[j]], dsem, add=True)
        return carry

    lax.fori_loop(0, K, fire, 0)

    def drain(j, carry):
        pltpu.make_async_copy(ones_v, deg_sh.at[idx_v.at[j]], dsem).wait()
        return carry

    lax.fori_loop(0, K, drain, 0)
    plsc.subcore_barrier()
    pltpu.sync_copy(deg_sh.at[pl.ds(s * RPT, RPT)], out.at[c, pl.ds(s * RPT, RPT)])


@functools.partial(
    pl.kernel,
    out_type=jax.ShapeDtypeStruct((NC, SCAT_ROWS, D), jnp.float32),
    mesh=_sc_mesh,
    scratch_types=[
        pltpu.VMEM((EPT_PAD,), jnp.int32),
        pltpu.VMEM((K, CH), jnp.int32),
        [pltpu.VMEM((CH, D), jnp.float32) for _ in range(NBUF)],
        pltpu.VMEM_SHARED((SCAT_ROWS, D), jnp.float32),
        [pltpu.SemaphoreType.DMA for _ in range(NBUF)],
    ],
)
def _edge_kernel(u, src2, dst3, zeros_blk, out, src_v, dst_v, bufs, scat_sh,
                 gsems):
    c = lax.axis_index("c")
    s = lax.axis_index("s")
    wid = c * NS + s
    for t in range(NZB):
        pltpu.sync_copy(zeros_blk, scat_sh.at[pl.ds(s * RPT + t * ZB, ZB)])
    pltpu.sync_copy(src2.at[wid], src_v)
    pltpu.sync_copy(dst3.at[wid], dst_v)
    plsc.subcore_barrier()

    for b in range(NBUF):
        pltpu.async_copy(u.at[src_v.at[pl.ds(b * CH, CH)]], bufs[b], gsems[b])

    def group(i, carry):
        for b in range(NBUF):
            j = i * NBUF + b
            pltpu.make_async_copy(
                u.at[src_v.at[pl.ds(j * CH, CH)]], bufs[b], gsems[b]).wait()
            pltpu.sync_copy(bufs[b], scat_sh.at[dst_v.at[j]], add=True)

            @pl.when(j + NBUF < K)
            def _():
                pltpu.async_copy(
                    u.at[src_v.at[pl.ds((j + NBUF) * CH, CH)]], bufs[b],
                    gsems[b])
        return carry

    lax.fori_loop(0, K // NBUF, group, 0)
    plsc.subcore_barrier()
    pltpu.sync_copy(scat_sh.at[pl.ds(s * RPT, RPT)], out.at[c, pl.ds(s * RPT, RPT)])


def _tc1_body(x_ref, w_ref, da_ref, db_ref, u_ref, dinv_ref):
    deg = da_ref[...] + db_ref[...] + 1.0
    dinv = lax.rsqrt(deg)
    xw = jnp.dot(x_ref[...], w_ref[...], preferred_element_type=jnp.float32)
    u_ref[...] = xw * dinv
    dinv_ref[...] = dinv


def _tc1(x, W1, dega, degb):
    return pl.pallas_call(
        _tc1_body,
        grid=(NBLK,),
        in_specs=[
            pl.BlockSpec((RB, D), lambda i: (i, 0)),
            pl.BlockSpec((D, D), lambda i: (0, 0)),
            pl.BlockSpec((RB, 1), lambda i: (i, 0)),
            pl.BlockSpec((RB, 1), lambda i: (i, 0)),
        ],
        out_specs=[
            pl.BlockSpec((RB, D), lambda i: (i, 0)),
            pl.BlockSpec((RB, 1), lambda i: (i, 0)),
        ],
        out_shape=[
            jax.ShapeDtypeStruct((N, D), jnp.float32),
            jax.ShapeDtypeStruct((N, 1), jnp.float32),
        ],
    )(x, W1, dega, degb)


def _tc2_body(sa_ref, sb_ref, u_ref, dinv_ref, b_ref, w_ref, h_ref, u2_ref):
    dv = dinv_ref[...]
    h = jnp.maximum((sa_ref[...] + sb_ref[...] + u_ref[...]) * dv + b_ref[...], 0.0)
    h_ref[...] = h
    u2_ref[...] = jnp.dot(h, w_ref[...], preferred_element_type=jnp.float32) * dv


def _tc2(sa, sb, u1, dinv, b1, W2):
    return pl.pallas_call(
        _tc2_body,
        grid=(NBLK,),
        in_specs=[
            pl.BlockSpec((RB, D), lambda i: (i, 0)),
            pl.BlockSpec((RB, D), lambda i: (i, 0)),
            pl.BlockSpec((RB, D), lambda i: (i, 0)),
            pl.BlockSpec((RB, 1), lambda i: (i, 0)),
            pl.BlockSpec((1, D), lambda i: (0, 0)),
            pl.BlockSpec((D, D), lambda i: (0, 0)),
        ],
        out_specs=[
            pl.BlockSpec((RB, D), lambda i: (i, 0)),
            pl.BlockSpec((RB, D), lambda i: (i, 0)),
        ],
        out_shape=[
            jax.ShapeDtypeStruct((N, D), jnp.float32),
            jax.ShapeDtypeStruct((N, D), jnp.float32),
        ],
    )(sa, sb, u1, dinv, b1, W2)


def _tc3_body(sa_ref, sb_ref, u2_ref, dinv_ref, b2_ref, h1_ref, batch_ref,
              wout_ref, bout_ref, out_ref):
    h2 = ((sa_ref[...] + sb_ref[...] + u2_ref[...]) * dinv_ref[...]
          + b2_ref[...] + h1_ref[...])
    bt = batch_ref[...]
    ids = lax.broadcasted_iota(jnp.int32, (G, N), 0)
    p = (ids == jnp.broadcast_to(bt, (G, N))).astype(jnp.float32)
    sums = jnp.dot(p, h2, preferred_element_type=jnp.float32)
    cnt = jnp.sum(p, axis=1, keepdims=True)
    gm = sums / jnp.maximum(cnt, 1.0)
    out_ref[...] = lax.dot_general(
        gm, wout_ref[...], (((1,), (1,)), ((), ())),
        preferred_element_type=jnp.float32) + bout_ref[...]


def _tc3(sa, sb, u2, dinv, b2, h1, batch2d, Wout, bout):
    return pl.pallas_call(
        _tc3_body,
        out_shape=jax.ShapeDtypeStruct((G, D), jnp.float32),
    )(sa, sb, u2, dinv, b2, h1, batch2d, Wout, bout)


def kernel(x, edge_index, batch, W1, b1, W2, b2, Wout, bout):
    src = edge_index[0].astype(jnp.int32)
    dst = edge_index[1].astype(jnp.int32)
    pad = E_PAD - E
    src2 = jnp.concatenate([src, jnp.zeros((pad,), jnp.int32)]).reshape(NW, EPT_PAD)
    dst3 = jnp.concatenate([dst, jnp.full((pad,), TRASH, jnp.int32)]).reshape(NW, K, CH)
    ones_col = jnp.ones((CH, DEGW), jnp.float32)
    zeros_col = jnp.zeros((RPT, DEGW), jnp.float32)
    zeros_blk = jnp.zeros((ZB, D), jnp.float32)

    deg = _deg_kernel(dst3, ones_col, zeros_col)
    dega = deg[0, :N, 0:1]
    degb = deg[1, :N, 0:1]
    u1, dinv = _tc1(x, W1, dega, degb)
    s1 = _edge_kernel(u1, src2, dst3, zeros_blk)
    h1, u2 = _tc2(s1[0, :N], s1[1, :N], u1, dinv, b1.reshape(1, D), W2)
    s2 = _edge_kernel(u2, src2, dst3, zeros_blk)
    out = _tc3(s2[0, :N], s2[1, :N], u2, dinv, b2.reshape(1, D), h1,
               batch.reshape(1, N).astype(jnp.int32), Wout,
               bout.reshape(1, D))
    return out
